# baseline (speedup 1.0000x reference)
import functools

import jax
import jax.numpy as jnp
from jax import lax
from jax.experimental import pallas as pl
from jax.experimental.pallas import tpu as pltpu
from jax.experimental.pallas import tpu_sc as plsc

D = 768
L = 16
NCH = D // L
CHUNK = 64


def _sc_embed_ln(xf, tok_table, pos_table, gamma, beta, *, n_tok, seq_len):
    info = plsc.get_sparse_core_info()
    nw = info.num_cores * info.num_subcores
    per_w = n_tok // nw
    n_chunks = per_w // CHUNK

    mesh = plsc.VectorSubcoreMesh(core_axis_name="c", subcore_axis_name="s")

    @functools.partial(
        pl.kernel,
        mesh=mesh,
        out_type=jax.ShapeDtypeStruct((n_tok, D), jnp.float32),
        compiler_params=pltpu.CompilerParams(needs_layout_passes=False),
        scratch_types=[
            pltpu.VMEM((CHUNK,), jnp.int32),
            pltpu.VMEM((CHUNK, D), jnp.float32),
            pltpu.VMEM((CHUNK, D), jnp.float32),
            pltpu.VMEM((D,), jnp.float32),
            pltpu.VMEM((D,), jnp.float32),
            pltpu.SemaphoreType.DMA,
        ],
    )
    def k(x_hbm, tok_hbm, pos_hbm, gamma_hbm, beta_hbm, out_hbm,
          idx_v, rows_v, pos_v, gamma_v, beta_v, sem):
        wid = lax.axis_index("s") * info.num_cores + lax.axis_index("c")
        base = wid * per_w
        pltpu.sync_copy(gamma_hbm, gamma_v)
        pltpu.sync_copy(beta_hbm, beta_v)

        def chunk_body(ci, carry):
            tbase = base + ci * CHUNK
            sbase = lax.rem(tbase, seq_len)
            pltpu.sync_copy(x_hbm.at[pl.ds(tbase, CHUNK)], idx_v)
            pltpu.async_copy(tok_hbm.at[idx_v], rows_v, sem).wait()
            pltpu.sync_copy(pos_hbm.at[pl.ds(sbase, CHUNK)], pos_v)

            def tok_body(t, tc):
                acc = jnp.zeros((L,), jnp.float32)
                acc2 = jnp.zeros((L,), jnp.float32)
                for j in range(NCH):
                    h = rows_v[t, pl.ds(j * L, L)] + pos_v[t, pl.ds(j * L, L)]
                    rows_v[t, pl.ds(j * L, L)] = h
                    acc = acc + h
                    acc2 = acc2 + h * h
                mean = jnp.sum(acc) * (1.0 / D)
                var = jnp.sum(acc2) * (1.0 / D) - mean * mean
                meanv = jnp.full((L,), mean, dtype=jnp.float32)
                tv = jnp.full((L,), var + 1e-5, dtype=jnp.float32)
                yi = jnp.full((L,), 0x5F3759DF, dtype=jnp.int32) - (
                    plsc.bitcast(tv, jnp.int32) >> 1)
                y = plsc.bitcast(yi, jnp.float32)
                for _ in range(4):
                    y = y * (1.5 - 0.5 * tv * y * y)
                for j in range(NCH):
                    h = rows_v[t, pl.ds(j * L, L)]
                    g = gamma_v[pl.ds(j * L, L)]
                    b = beta_v[pl.ds(j * L, L)]
                    rows_v[t, pl.ds(j * L, L)] = (h - meanv) * y * g + b
                return tc

            lax.fori_loop(0, CHUNK, tok_body, 0)
            pltpu.sync_copy(rows_v, out_hbm.at[pl.ds(tbase, CHUNK)])
            return carry

        lax.fori_loop(0, n_chunks, chunk_body, 0)

    return k(xf, tok_table, pos_table, gamma, beta)


def kernel(x, tok_table, pos_table, gamma, beta):
    b, s = x.shape
    xf = x.reshape(b * s).astype(jnp.int32)
    out = _sc_embed_ln(xf, tok_table, pos_table, gamma, beta,
                       n_tok=b * s, seq_len=s)
    return out.reshape(b, s, D)


# speedup vs baseline: 2.2088x; 2.2088x over previous
import functools

import jax
import jax.numpy as jnp
from jax import lax
from jax.experimental import pallas as pl
from jax.experimental.pallas import tpu as pltpu
from jax.experimental.pallas import tpu_sc as plsc

D = 768
CHUNK = 64
TC_BLK = 512


def _sc_gather(xf, tok_table, *, n_tok):
    info = plsc.get_sparse_core_info()
    nw = info.num_cores * info.num_subcores
    per_w = n_tok // nw
    n_chunks = per_w // CHUNK

    mesh = plsc.VectorSubcoreMesh(core_axis_name="c", subcore_axis_name="s")

    @functools.partial(
        pl.kernel,
        mesh=mesh,
        out_type=jax.ShapeDtypeStruct((n_tok, D), jnp.float32),
        compiler_params=pltpu.CompilerParams(needs_layout_passes=False),
        scratch_types=[
            pltpu.VMEM((n_chunks, CHUNK), jnp.int32),
            pltpu.VMEM((CHUNK, D), jnp.float32),
            pltpu.VMEM((CHUNK, D), jnp.float32),
            pltpu.SemaphoreType.DMA,
            pltpu.SemaphoreType.DMA,
            pltpu.SemaphoreType.DMA,
        ],
    )
    def k(x_hbm, tok_hbm, out_hbm, idx_v, buf0, buf1, gsem, osem0, osem1):
        wid = lax.axis_index("s") * info.num_cores + lax.axis_index("c")
        base = wid * per_w
        pltpu.sync_copy(x_hbm.at[wid], idx_v)

        bufs = (buf0, buf1)
        osems = (osem0, osem1)
        pltpu.async_copy(tok_hbm.at[idx_v.at[0]], buf0, gsem).wait()
        for ci in range(1, n_chunks + 1):
            cur = (ci - 1) % 2
            nxt = ci % 2
            gn = None
            if ci < n_chunks:
                gn = pltpu.async_copy(tok_hbm.at[idx_v.at[ci]], bufs[nxt], gsem)
            out = pltpu.async_copy(
                bufs[cur], out_hbm.at[pl.ds(base + (ci - 1) * CHUNK, CHUNK)],
                osems[cur])
            if gn is not None:
                gn.wait()
            out.wait()

    return k(xf.reshape(nw, n_chunks, CHUNK), tok_table)


def _tc_add_ln(rows, pos_table, gamma2d, beta2d, *, n_tok, seq_len):
    n_blk = n_tok // TC_BLK
    pos_blocks = seq_len // TC_BLK

    def body(r_ref, p_ref, g_ref, b_ref, o_ref):
        h = r_ref[...] + p_ref[...]
        mean = jnp.mean(h, axis=-1, keepdims=True)
        c = h - mean
        var = jnp.mean(c * c, axis=-1, keepdims=True)
        inv = lax.rsqrt(var + 1e-5)
        o_ref[...] = c * inv * g_ref[...] + b_ref[...]

    return pl.pallas_call(
        body,
        grid=(n_blk,),
        in_specs=[
            pl.BlockSpec((TC_BLK, D), lambda i: (i, 0)),
            pl.BlockSpec((TC_BLK, D), lambda i: (i % pos_blocks, 0)),
            pl.BlockSpec((1, D), lambda i: (0, 0)),
            pl.BlockSpec((1, D), lambda i: (0, 0)),
        ],
        out_specs=pl.BlockSpec((TC_BLK, D), lambda i: (i, 0)),
        out_shape=jax.ShapeDtypeStruct((n_tok, D), jnp.float32),
    )(rows, pos_table, gamma2d, beta2d)


def kernel(x, tok_table, pos_table, gamma, beta):
    b, s = x.shape
    n_tok = b * s
    xf = x.reshape(n_tok).astype(jnp.int32)
    rows = _sc_gather(xf, tok_table, n_tok=n_tok)
    out = _tc_add_ln(rows, pos_table, gamma.reshape(1, D), beta.reshape(1, D),
                     n_tok=n_tok, seq_len=s)
    return out.reshape(b, s, D)


# speedup vs baseline: 2.2796x; 1.0320x over previous
import functools

import jax
import jax.numpy as jnp
from jax import lax
from jax.experimental import pallas as pl
from jax.experimental.pallas import tpu as pltpu
from jax.experimental.pallas import tpu_sc as plsc

D = 768
CHUNK = 64
TC_BLK = 512


def _sc_gather(xf, tok_table, *, n_tok):
    info = plsc.get_sparse_core_info()
    nw = info.num_cores * info.num_subcores
    per_w = n_tok // nw
    n_chunks = per_w // CHUNK

    mesh = plsc.VectorSubcoreMesh(core_axis_name="c", subcore_axis_name="s")

    @functools.partial(
        pl.kernel,
        mesh=mesh,
        out_type=jax.ShapeDtypeStruct((n_tok, D), jnp.float32),
        compiler_params=pltpu.CompilerParams(needs_layout_passes=False),
        scratch_types=[
            pltpu.VMEM((n_chunks, CHUNK), jnp.int32),
            pltpu.VMEM((CHUNK, D), jnp.float32),
            pltpu.VMEM((CHUNK, D), jnp.float32),
            pltpu.SemaphoreType.DMA,
            pltpu.SemaphoreType.DMA,
            pltpu.SemaphoreType.DMA,
        ],
    )
    def k(x_hbm, tok_hbm, out_hbm, idx_v, buf0, buf1, gsem, osem0, osem1):
        wid = lax.axis_index("s") * info.num_cores + lax.axis_index("c")
        base = wid * per_w
        pltpu.sync_copy(x_hbm.at[wid], idx_v)

        bufs = (buf0, buf1)
        osems = (osem0, osem1)
        pltpu.async_copy(tok_hbm.at[idx_v.at[0]], buf0, gsem).wait()
        for ci in range(1, n_chunks + 1):
            cur = (ci - 1) % 2
            nxt = ci % 2
            gn = None
            if ci < n_chunks:
                gn = pltpu.async_copy(tok_hbm.at[idx_v.at[ci]], bufs[nxt], gsem)
            out = pltpu.async_copy(
                bufs[cur], out_hbm.at[pl.ds(base + (ci - 1) * CHUNK, CHUNK)],
                osems[cur])
            if gn is not None:
                gn.wait()
            out.wait()

    return k(xf.reshape(nw, n_chunks, CHUNK), tok_table)


def _tc_add_ln(rows, pos_table, gamma2d, beta2d, *, n_tok, seq_len):
    n_blk = n_tok // TC_BLK
    pos_blocks = seq_len // TC_BLK

    def body(r_ref, p_ref, g_ref, b_ref, o_ref):
        h = r_ref[...] + p_ref[...]
        mean = jnp.mean(h, axis=-1, keepdims=True)
        c = h - mean
        var = jnp.mean(c * c, axis=-1, keepdims=True)
        inv = lax.rsqrt(var + 1e-5)
        o_ref[...] = c * inv * g_ref[...] + b_ref[...]

    n_batch = n_blk // pos_blocks
    return pl.pallas_call(
        body,
        grid=(pos_blocks, n_batch),
        in_specs=[
            pl.BlockSpec((TC_BLK, D), lambda i, j: (j * pos_blocks + i, 0)),
            pl.BlockSpec((TC_BLK, D), lambda i, j: (i, 0)),
            pl.BlockSpec((1, D), lambda i, j: (0, 0)),
            pl.BlockSpec((1, D), lambda i, j: (0, 0)),
        ],
        out_specs=pl.BlockSpec((TC_BLK, D), lambda i, j: (j * pos_blocks + i, 0)),
        out_shape=jax.ShapeDtypeStruct((n_tok, D), jnp.float32),
    )(rows, pos_table, gamma2d, beta2d)


def kernel(x, tok_table, pos_table, gamma, beta):
    b, s = x.shape
    n_tok = b * s
    xf = x.reshape(n_tok).astype(jnp.int32)
    rows = _sc_gather(xf, tok_table, n_tok=n_tok)
    out = _tc_add_ln(rows, pos_table, gamma.reshape(1, D), beta.reshape(1, D),
                     n_tok=n_tok, seq_len=s)
    return out.reshape(b, s, D)
